# Initial kernel scaffold; baseline (speedup 1.0000x reference)
import jax
import jax.numpy as jnp
from jax.experimental import pallas as pl


def kernel(features, coarse_map):
    raise NotImplementedError("write your pallas kernel here")


# speedup vs baseline: 21.0529x; 21.0529x over previous
import functools

import jax
import jax.numpy as jnp
from jax import lax
from jax.experimental import pallas as pl
from jax.experimental.pallas import tpu as pltpu
from jax.experimental.pallas import tpu_sc as plsc

B, N, C = 8, 4096, 256
NPOINT, NSAMPLE = 1024, 32

NC, NS, L = 2, 16, 16
NW = NC * NS
PPW = (B * NPOINT) // NW
CHUNK = 4
ROWS = CHUNK * NSAMPLE
OUTCHUNK = 8
NOUT = PPW // OUTCHUNK
CG = C // L

_mesh = plsc.VectorSubcoreMesh(core_axis_name="c", subcore_axis_name="s")


@functools.partial(
    pl.kernel,
    out_type=jax.ShapeDtypeStruct((B * NPOINT, C), jnp.float32),
    mesh=_mesh,
    scratch_types=[
        pltpu.VMEM((PPW * NSAMPLE,), jnp.int32),
        pltpu.VMEM((ROWS, C), jnp.float32),
        pltpu.VMEM((OUTCHUNK, C), jnp.float32),
        pltpu.SemaphoreType.DMA,
    ],
)
def _pool(feat_hbm, idx_hbm, out_hbm, idx_v, rows_v, out_v, sem):
    wid = lax.axis_index("s") * NC + lax.axis_index("c")
    base = wid * PPW
    boff = (base // NPOINT) * N

    pltpu.sync_copy(idx_hbm.at[pl.ds(base * NSAMPLE, PPW * NSAMPLE)], idx_v)

    def _rebase(j, _):
        v = idx_v[pl.ds(j * L, L)]
        idx_v[pl.ds(j * L, L)] = v + boff
        return _
    lax.fori_loop(0, (PPW * NSAMPLE) // L, _rebase, None)

    def _outgroup(k, _):
        for h in range(OUTCHUNK // CHUNK):
            c = k * (OUTCHUNK // CHUNK) + h
            cpy = pltpu.async_copy(
                feat_hbm.at[idx_v.at[pl.ds(c * ROWS, ROWS)]], rows_v, sem)
            cpy.wait()
            for p in range(CHUNK):
                def _colgroup(g, _):
                    acc = rows_v[p * NSAMPLE, pl.ds(g * L, L)]
                    for s in range(1, NSAMPLE):
                        acc = jnp.maximum(
                            acc, rows_v[p * NSAMPLE + s, pl.ds(g * L, L)])
                    out_v[h * CHUNK + p, pl.ds(g * L, L)] = acc
                    return _
                lax.fori_loop(0, CG, _colgroup, None)
        pltpu.sync_copy(out_v, out_hbm.at[pl.ds(base + k * OUTCHUNK, OUTCHUNK)])
        return _
    lax.fori_loop(0, NOUT, _outgroup, None)


def kernel(features, coarse_map):
    feat_flat = features.reshape(B * N, C)
    idx_flat = coarse_map.reshape(B * NPOINT * NSAMPLE)
    out = _pool(feat_flat, idx_flat)
    return out.reshape(B, NPOINT, C)


# speedup vs baseline: 35.5862x; 1.6903x over previous
import functools

import jax
import jax.numpy as jnp
from jax import lax
from jax.experimental import pallas as pl
from jax.experimental.pallas import tpu as pltpu
from jax.experimental.pallas import tpu_sc as plsc

B, N, C = 8, 4096, 256
NPOINT, NSAMPLE = 1024, 32

NC, NS, L = 2, 16, 16
NW = NC * NS
PPW = (B * NPOINT) // NW
CHUNK = 4
ROWS = CHUNK * NSAMPLE
OUTCHUNK = 8
NITER = PPW // (2 * OUTCHUNK)
CG = C // L

_mesh = plsc.VectorSubcoreMesh(core_axis_name="c", subcore_axis_name="s")


@functools.partial(
    pl.kernel,
    out_type=jax.ShapeDtypeStruct((B * NPOINT, C), jnp.float32),
    mesh=_mesh,
    scratch_types=[
        pltpu.VMEM((PPW * NSAMPLE,), jnp.int32),
        pltpu.VMEM((ROWS, C), jnp.float32),
        pltpu.VMEM((ROWS, C), jnp.float32),
        pltpu.VMEM((OUTCHUNK, C), jnp.float32),
        pltpu.VMEM((OUTCHUNK, C), jnp.float32),
        pltpu.SemaphoreType.DMA,
        pltpu.SemaphoreType.DMA,
        pltpu.SemaphoreType.DMA,
        pltpu.SemaphoreType.DMA,
    ],
)
def _pool(feat_hbm, idx_hbm, out_hbm, idx_v, rows_a, rows_b, out_a, out_b,
          sem_a, sem_b, sem_oa, sem_ob):
    wid = lax.axis_index("s") * NC + lax.axis_index("c")
    base = wid * PPW
    boff = (base // NPOINT) * N

    pltpu.sync_copy(idx_hbm.at[pl.ds(base * NSAMPLE, PPW * NSAMPLE)], idx_v)

    def _rebase(j, carry):
        idx_v[pl.ds(j * L, L)] = idx_v[pl.ds(j * L, L)] + boff
        return carry
    lax.fori_loop(0, (PPW * NSAMPLE) // L, _rebase, None)

    def _gather(c, rows_v, sem):
        return pltpu.make_async_copy(
            feat_hbm.at[idx_v.at[pl.ds(c * ROWS, ROWS)]], rows_v, sem)

    def _owrite(k8, out_v, sem):
        return pltpu.make_async_copy(
            out_v, out_hbm.at[pl.ds(base + k8 * OUTCHUNK, OUTCHUNK)], sem)

    def _compute(rows_v, out_v, h):
        def _colgroup(g, carry):
            for p in range(CHUNK):
                r0 = p * NSAMPLE
                accs = [rows_v[r0 + t, pl.ds(g * L, L)] for t in range(4)]
                for s in range(4, NSAMPLE, 4):
                    for t in range(4):
                        accs[t] = jnp.maximum(
                            accs[t], rows_v[r0 + s + t, pl.ds(g * L, L)])
                acc = jnp.maximum(jnp.maximum(accs[0], accs[1]),
                                  jnp.maximum(accs[2], accs[3]))
                out_v[h * CHUNK + p, pl.ds(g * L, L)] = acc
            return carry
        lax.fori_loop(0, CG, _colgroup, None)

    _gather(0, rows_a, sem_a).start()
    _gather(1, rows_b, sem_b).start()

    def _step(k, carry):
        c0 = k * 4
        k8 = k * 2
        _gather(c0, rows_a, sem_a).wait()

        @pl.when(k > 0)
        def _():
            _owrite(k8 - 2, out_a, sem_oa).wait()
        _compute(rows_a, out_a, 0)
        _gather(c0 + 2, rows_a, sem_a).start()

        _gather(c0 + 1, rows_b, sem_b).wait()
        _compute(rows_b, out_a, 1)
        _gather(c0 + 3, rows_b, sem_b).start()
        _owrite(k8, out_a, sem_oa).start()

        _gather(c0 + 2, rows_a, sem_a).wait()

        @pl.when(k > 0)
        def _():
            _owrite(k8 - 1, out_b, sem_ob).wait()
        _compute(rows_a, out_b, 0)

        @pl.when(k < NITER - 1)
        def _():
            _gather(c0 + 4, rows_a, sem_a).start()

        _gather(c0 + 3, rows_b, sem_b).wait()
        _compute(rows_b, out_b, 1)

        @pl.when(k < NITER - 1)
        def _():
            _gather(c0 + 5, rows_b, sem_b).start()
        _owrite(k8 + 1, out_b, sem_ob).start()
        return carry

    lax.fori_loop(0, NITER, _step, None)
    _owrite(2 * NITER - 2, out_a, sem_oa).wait()
    _owrite(2 * NITER - 1, out_b, sem_ob).wait()


def kernel(features, coarse_map):
    feat_flat = features.reshape(B * N, C)
    idx_flat = coarse_map.reshape(B * NPOINT * NSAMPLE)
    out = _pool(feat_flat, idx_flat)
    return out.reshape(B, NPOINT, C)
